# baseline (speedup 1.0000x reference)
import functools
import math

import jax
import jax.numpy as jnp
from jax.experimental import pallas as pl
from jax.experimental.pallas import tpu as pltpu

R_RATIO = 0.95
MARGIN = 0.5
_PREC = jax.lax.Precision.DEFAULT


def _iso_body(n_blk_ref, x_blk_ref, n_all_ref, x_all_ref, z_ref):
    nb = n_blk_ref[0, 0, :]
    na = n_all_ref[0, 0, :]
    xi = x_blk_ref[0] / jnp.clip(nb[:, None], 1e-12, None)
    xa = x_all_ref[0] / jnp.clip(na[:, None], 1e-12, None)
    T = xa.shape[0]
    sim = jax.lax.dot_general(xi, xa, (((1,), (1,)), ((), ())),
                              preferred_element_type=jnp.float32,
                              precision=_PREC)
    cnt = jnp.sum(jnp.where(sim > MARGIN, 1.0, -1.0), axis=1)
    ssum = jnp.sum(sim, axis=1)
    z_ref[0, 0, :] = cnt / T + ssum / T


def _iso_pass(x, n3, block_i=512):
    B, T, C = x.shape
    z = pl.pallas_call(
        _iso_body,
        grid=(B, T // block_i),
        in_specs=[
            pl.BlockSpec((1, 1, block_i), lambda b, i: (b, 0, i)),
            pl.BlockSpec((1, block_i, C), lambda b, i: (b, i, 0)),
            pl.BlockSpec((1, 1, T), lambda b, i: (b, 0, 0)),
            pl.BlockSpec((1, T, C), lambda b, i: (b, 0, 0)),
        ],
        out_specs=pl.BlockSpec((1, 1, block_i), lambda b, i: (b, 0, i)),
        out_shape=jax.ShapeDtypeStruct((B, 1, T), jnp.float32),
    )(n3, x, n3, x)
    return z


def _rank_body(iso_col_ref, iso_row_ref, rank_ref, *, block_i):
    ic = iso_col_ref[0]
    ir = iso_row_ref[0]
    BI = ic.shape[0]
    T = ir.shape[1]
    ib = pl.program_id(1)
    ii = jax.lax.broadcasted_iota(jnp.int32, (BI, T), 0).astype(jnp.float32) + ib * block_i
    jj = jax.lax.broadcasted_iota(jnp.int32, (BI, T), 1).astype(jnp.float32)
    less = ir < ic
    eq = ir == ic
    take = jnp.logical_or(less, jnp.logical_and(eq, jj < ii))
    rank_ref[0] = jnp.sum(jnp.where(take, 1.0, 0.0), axis=1, keepdims=True)


def _rank_pass(iso_col, iso_row, block_i=256):
    B, T, _ = iso_col.shape
    return pl.pallas_call(
        functools.partial(_rank_body, block_i=block_i),
        grid=(B, T // block_i),
        in_specs=[
            pl.BlockSpec((1, block_i, 1), lambda b, i: (b, i, 0)),
            pl.BlockSpec((1, 1, T), lambda b, i: (b, 0, 0)),
        ],
        out_specs=pl.BlockSpec((1, block_i, 1), lambda b, i: (b, i, 0)),
        out_shape=jax.ShapeDtypeStruct((B, T, 1), jnp.float32),
    )(iso_col, iso_row)


def _select_body(rank_col_ref, rank_row_ref, n_col_ref, x_ref, outpos_ref,
                 *, r, n_prot, kpad):
    T, C = x_ref[0].shape
    rank_row = rank_row_ref[0]
    xn = x_ref[0] / jnp.clip(n_col_ref[0], 1e-12, None)

    kk = jax.lax.broadcasted_iota(jnp.int32, (kpad, T), 0).astype(jnp.float32)
    kvalid = kk < r
    rr = jnp.broadcast_to(rank_row, (kpad, T))
    ea = jnp.where(jnp.logical_and(rr == 2.0 * kk, kvalid), 1.0, 0.0)
    eb = jnp.where(jnp.logical_and(rr == 2.0 * kk + 1.0, kvalid), 1.0, 0.0)

    xa = jax.lax.dot_general(ea, xn, (((1,), (0,)), ((), ())),
                             preferred_element_type=jnp.float32, precision=_PREC)
    xb = jax.lax.dot_general(eb, xn, (((1,), (0,)), ((), ())),
                             preferred_element_type=jnp.float32, precision=_PREC)
    scores = jax.lax.dot_general(xa, xb, (((1,), (1,)), ((), ())),
                                 preferred_element_type=jnp.float32,
                                 precision=_PREC)
    jjk = jax.lax.broadcasted_iota(jnp.int32, (kpad, kpad), 1).astype(jnp.float32)
    scores_m = jnp.where(jjk < r, scores, -3.0)
    mx = jnp.max(scores_m, axis=1, keepdims=True)
    dst = jnp.min(jnp.where(scores_m == mx, jjk, float(kpad)), axis=1,
                  keepdims=True)

    dst_tok = jnp.sum(ea * dst, axis=0, keepdims=True)

    rank_col = rank_col_ref[0]
    BJ = 256
    c = jnp.zeros((1, T), jnp.float32)
    for jb in range(T // BJ):
        prot_col = jnp.where(rank_col[jb * BJ:(jb + 1) * BJ, :] >= 2.0 * r,
                             1.0, 0.0)
        jidx = jax.lax.broadcasted_iota(jnp.int32, (BJ, T), 0).astype(jnp.float32) + jb * BJ
        iidx = jax.lax.broadcasted_iota(jnp.int32, (BJ, T), 1).astype(jnp.float32)
        c = c + jnp.sum(jnp.where(jidx < iidx, prot_col, 0.0), axis=0,
                        keepdims=True)

    is_merged = rank_row < 2.0 * r
    is_b = jnp.logical_and(is_merged,
                           rank_row - 2.0 * jnp.floor(rank_row * 0.5) == 1.0)
    pos_b = n_prot + (rank_row - 1.0) * 0.5
    pos_a = n_prot + dst_tok
    outpos = jnp.where(is_merged, jnp.where(is_b, pos_b, pos_a), c)
    outpos_ref[0] = outpos


def _select_pass(rank_col, rank_row, n_col, x, r, n_prot, kpad=128):
    B, T, C = x.shape
    return pl.pallas_call(
        functools.partial(_select_body, r=r, n_prot=n_prot, kpad=kpad),
        grid=(B,),
        in_specs=[
            pl.BlockSpec((1, T, 1), lambda b: (b, 0, 0)),
            pl.BlockSpec((1, 1, T), lambda b: (b, 0, 0)),
            pl.BlockSpec((1, T, 1), lambda b: (b, 0, 0)),
            pl.BlockSpec((1, T, C), lambda b: (b, 0, 0)),
        ],
        out_specs=pl.BlockSpec((1, 1, T), lambda b: (b, 0, 0)),
        out_shape=jax.ShapeDtypeStruct((B, 1, T), jnp.float32),
    )(rank_col, rank_row, n_col, x)


def _merge_body(outpos_ref, w_ref, w_col_ref, x_ref, xrec_ref, size_ref, *, block_p):
    outpos = outpos_ref[0]
    w = w_ref[0]
    w_col = w_col_ref[0]
    x = x_ref[0]
    T = x.shape[0]
    pb = pl.program_id(1)
    pcol = jax.lax.broadcasted_iota(jnp.int32, (block_p, T), 0).astype(jnp.float32) + pb * block_p
    sel = jnp.broadcast_to(outpos, (block_p, T)) == pcol
    m01 = jnp.where(sel, 1.0, 0.0)
    y = x * w_col
    y_hi = y.astype(jnp.bfloat16).astype(jnp.float32)
    y_lo = y - y_hi
    acc = (jax.lax.dot_general(m01, y_hi, (((1,), (0,)), ((), ())),
                               preferred_element_type=jnp.float32,
                               precision=_PREC)
           + jax.lax.dot_general(m01, y_lo, (((1,), (0,)), ((), ())),
                                 preferred_element_type=jnp.float32,
                                 precision=_PREC))
    size = jnp.sum(jnp.where(sel, w, 0.0), axis=1, keepdims=True)
    xrec_ref[0] = acc / size
    size_ref[0] = size


def _merge_pass(outpos, iso_row, iso_col, x, n_out, block_p=256):
    B, T, C = x.shape
    n_pad = ((n_out + block_p - 1) // block_p) * block_p
    xrec, size = pl.pallas_call(
        functools.partial(_merge_body, block_p=block_p),
        grid=(B, n_pad // block_p),
        in_specs=[
            pl.BlockSpec((1, 1, T), lambda b, p: (b, 0, 0)),
            pl.BlockSpec((1, 1, T), lambda b, p: (b, 0, 0)),
            pl.BlockSpec((1, T, 1), lambda b, p: (b, 0, 0)),
            pl.BlockSpec((1, T, C), lambda b, p: (b, 0, 0)),
        ],
        out_specs=[
            pl.BlockSpec((1, block_p, C), lambda b, p: (b, p, 0)),
            pl.BlockSpec((1, block_p, 1), lambda b, p: (b, p, 0)),
        ],
        out_shape=[
            jax.ShapeDtypeStruct((B, n_pad, C), jnp.float32),
            jax.ShapeDtypeStruct((B, n_pad, 1), jnp.float32),
        ],
    )(outpos, iso_row, iso_col, x)
    return xrec[:, :n_out, :], size[:, :n_out, :]


def kernel(x):
    B, T, C = x.shape
    r = math.floor(T - T * R_RATIO)
    n_prot = T - 2 * r
    n_out = n_prot + r

    nrm = jnp.linalg.norm(x, axis=-1, keepdims=True)
    n3 = nrm[..., 0].reshape(B, 1, T)
    z = _iso_pass(x, n3)

    iso = 1.0 - jax.nn.softmax(z[:, 0, :], axis=-1)
    iso_row = iso.reshape(B, 1, T)
    iso_col = iso.reshape(B, T, 1)

    rank_col = _rank_pass(iso_col, iso_row)
    rank_row = rank_col.reshape(B, 1, T)
    outpos = _select_pass(rank_col, rank_row, nrm, x, r, n_prot)
    xrec, size = _merge_pass(outpos, iso_row, iso_col, x, n_out)
    return xrec, size


# speedup vs baseline: 1.5308x; 1.5308x over previous
import functools
import math

import jax
import jax.numpy as jnp
from jax.experimental import pallas as pl
from jax.experimental.pallas import tpu as pltpu

R_RATIO = 0.95
MARGIN = 0.5
_PREC = jax.lax.Precision.DEFAULT


def _iota_f32(shape, dim):
    return jax.lax.broadcasted_iota(jnp.int32, shape, dim).astype(jnp.float32)


def _iso_body(n_blk_ref, n_all_ref, x_all_ref, z_ref, *, block_i):
    ib = pl.program_id(1)
    nb = n_blk_ref[0, 0, :]
    na = n_all_ref[0, 0, :]
    xa = x_all_ref[0] / jnp.clip(na[:, None], 1e-12, None)
    T = xa.shape[0]
    xi = x_all_ref[0, pl.ds(ib * block_i, block_i), :] / jnp.clip(nb[:, None], 1e-12, None)
    sim = jax.lax.dot_general(xi, xa, (((1,), (1,)), ((), ())),
                              preferred_element_type=jnp.float32,
                              precision=_PREC)
    cnt = jnp.sum(jnp.where(sim > MARGIN, 1.0, -1.0), axis=1)
    ssum = jnp.sum(sim, axis=1)
    z_ref[0, 0, :] = cnt / T + ssum / T


def _iso_pass(x, n3, block_i=512):
    B, T, C = x.shape
    z = pl.pallas_call(
        functools.partial(_iso_body, block_i=block_i),
        grid=(B, T // block_i),
        in_specs=[
            pl.BlockSpec((1, 1, block_i), lambda b, i: (b, 0, i)),
            pl.BlockSpec((1, 1, T), lambda b, i: (b, 0, 0)),
            pl.BlockSpec((1, T, C), lambda b, i: (b, 0, 0)),
        ],
        out_specs=pl.BlockSpec((1, 1, block_i), lambda b, i: (b, 0, i)),
        out_shape=jax.ShapeDtypeStruct((B, 1, T), jnp.float32),
    )(n3, n3, x)
    return z


def _rank_body(iso_col_ref, iso_row_ref, rank_ref, *, chunk):
    ir = iso_row_ref[0]
    T = ir.shape[1]
    jj = _iota_f32((chunk, T), 1)
    for cb in range(T // chunk):
        ic = iso_col_ref[0, pl.ds(cb * chunk, chunk), :]
        ii = _iota_f32((chunk, T), 0) + float(cb * chunk)
        take = jnp.logical_or(ir < ic,
                              jnp.logical_and(ir == ic, jj < ii))
        rank_ref[0, pl.ds(cb * chunk, chunk), :] = jnp.sum(
            jnp.where(take, 1.0, 0.0), axis=1, keepdims=True)


def _rank_pass(iso_col, iso_row, chunk=512):
    B, T, _ = iso_col.shape
    return pl.pallas_call(
        functools.partial(_rank_body, chunk=chunk),
        grid=(B,),
        in_specs=[
            pl.BlockSpec((1, T, 1), lambda b: (b, 0, 0)),
            pl.BlockSpec((1, 1, T), lambda b: (b, 0, 0)),
        ],
        out_specs=pl.BlockSpec((1, T, 1), lambda b: (b, 0, 0)),
        out_shape=jax.ShapeDtypeStruct((B, T, 1), jnp.float32),
    )(iso_col, iso_row)


def _select_body(rank_col_ref, rank_row_ref, n_col_ref, x_ref, outpos_ref,
                 *, r, n_prot, kpad):
    T, C = x_ref[0].shape
    rank_row = rank_row_ref[0]
    xn = x_ref[0] / jnp.clip(n_col_ref[0], 1e-12, None)

    kk = _iota_f32((kpad, T), 0)
    kvalid = kk < r
    rr = jnp.broadcast_to(rank_row, (kpad, T))
    ea = jnp.where(jnp.logical_and(rr == 2.0 * kk, kvalid), 1.0, 0.0)
    eb = jnp.where(jnp.logical_and(rr == 2.0 * kk + 1.0, kvalid), 1.0, 0.0)

    xa = jax.lax.dot_general(ea, xn, (((1,), (0,)), ((), ())),
                             preferred_element_type=jnp.float32, precision=_PREC)
    xb = jax.lax.dot_general(eb, xn, (((1,), (0,)), ((), ())),
                             preferred_element_type=jnp.float32, precision=_PREC)
    scores = jax.lax.dot_general(xa, xb, (((1,), (1,)), ((), ())),
                                 preferred_element_type=jnp.float32,
                                 precision=_PREC)
    jjk = _iota_f32((kpad, kpad), 1)
    scores_m = jnp.where(jjk < r, scores, -3.0)
    mx = jnp.max(scores_m, axis=1, keepdims=True)
    dst = jnp.min(jnp.where(scores_m == mx, jjk, float(kpad)), axis=1,
                  keepdims=True)

    dst_tok = jnp.sum(ea * dst, axis=0, keepdims=True)

    BJ = 512
    c = jnp.zeros((1, T), jnp.float32)
    for jb in range(T // BJ):
        prot_col = jnp.where(rank_col_ref[0, pl.ds(jb * BJ, BJ), :] >= 2.0 * r,
                             1.0, 0.0)
        jidx = _iota_f32((BJ, T), 0) + float(jb * BJ)
        iidx = _iota_f32((BJ, T), 1)
        c = c + jnp.sum(jnp.where(jidx < iidx, prot_col, 0.0), axis=0,
                        keepdims=True)

    is_merged = rank_row < 2.0 * r
    is_b = jnp.logical_and(is_merged,
                           rank_row - 2.0 * jnp.floor(rank_row * 0.5) == 1.0)
    pos_b = n_prot + (rank_row - 1.0) * 0.5
    pos_a = n_prot + dst_tok
    outpos = jnp.where(is_merged, jnp.where(is_b, pos_b, pos_a), c)
    outpos_ref[0] = outpos


def _select_pass(rank_col, rank_row, n_col, x, r, n_prot, kpad=128):
    B, T, C = x.shape
    return pl.pallas_call(
        functools.partial(_select_body, r=r, n_prot=n_prot, kpad=kpad),
        grid=(B,),
        in_specs=[
            pl.BlockSpec((1, T, 1), lambda b: (b, 0, 0)),
            pl.BlockSpec((1, 1, T), lambda b: (b, 0, 0)),
            pl.BlockSpec((1, T, 1), lambda b: (b, 0, 0)),
            pl.BlockSpec((1, T, C), lambda b: (b, 0, 0)),
        ],
        out_specs=pl.BlockSpec((1, 1, T), lambda b: (b, 0, 0)),
        out_shape=jax.ShapeDtypeStruct((B, 1, T), jnp.float32),
    )(rank_col, rank_row, n_col, x)


def _merge_body(outpos_ref, w_ref, w_col_ref, x_ref, xrec_ref, size_ref,
                *, block_p):
    outpos = outpos_ref[0]
    w = w_ref[0]
    w_col = w_col_ref[0]
    x = x_ref[0]
    T = x.shape[0]
    pb = pl.program_id(1)
    pcol = _iota_f32((block_p, T), 0) + pb * block_p
    sel = jnp.broadcast_to(outpos, (block_p, T)) == pcol
    m01 = jnp.where(sel, 1.0, 0.0)
    y = x * w_col
    y_hi = y.astype(jnp.bfloat16).astype(jnp.float32)
    y_lo = y - y_hi
    acc = (jax.lax.dot_general(m01, y_hi, (((1,), (0,)), ((), ())),
                               preferred_element_type=jnp.float32,
                               precision=_PREC)
           + jax.lax.dot_general(m01, y_lo, (((1,), (0,)), ((), ())),
                                 preferred_element_type=jnp.float32,
                                 precision=_PREC))
    size = jnp.sum(jnp.where(sel, w, 0.0), axis=1, keepdims=True)
    xrec_ref[0] = acc / size
    size_ref[0] = size


def _merge_pass(outpos, iso_row, iso_col, x, n_out, block_p=256):
    B, T, C = x.shape
    n_blocks = (n_out + block_p - 1) // block_p
    xrec, size = pl.pallas_call(
        functools.partial(_merge_body, block_p=block_p),
        grid=(B, n_blocks),
        in_specs=[
            pl.BlockSpec((1, 1, T), lambda b, p: (b, 0, 0)),
            pl.BlockSpec((1, 1, T), lambda b, p: (b, 0, 0)),
            pl.BlockSpec((1, T, 1), lambda b, p: (b, 0, 0)),
            pl.BlockSpec((1, T, C), lambda b, p: (b, 0, 0)),
        ],
        out_specs=[
            pl.BlockSpec((1, block_p, C), lambda b, p: (b, p, 0)),
            pl.BlockSpec((1, block_p, 1), lambda b, p: (b, p, 0)),
        ],
        out_shape=[
            jax.ShapeDtypeStruct((B, n_out, C), jnp.float32),
            jax.ShapeDtypeStruct((B, n_out, 1), jnp.float32),
        ],
    )(outpos, iso_row, iso_col, x)
    return xrec, size


def kernel(x):
    B, T, C = x.shape
    r = math.floor(T - T * R_RATIO)
    n_prot = T - 2 * r
    n_out = n_prot + r

    nrm = jnp.linalg.norm(x, axis=-1, keepdims=True)
    n3 = nrm[..., 0].reshape(B, 1, T)
    z = _iso_pass(x, n3)

    iso = 1.0 - jax.nn.softmax(z[:, 0, :], axis=-1)
    iso_row = iso.reshape(B, 1, T)
    iso_col = iso.reshape(B, T, 1)

    rank_col = _rank_pass(iso_col, iso_row)
    rank_row = rank_col.reshape(B, 1, T)
    outpos = _select_pass(rank_col, rank_row, nrm, x, r, n_prot)
    xrec, size = _merge_pass(outpos, iso_row, iso_col, x, n_out)
    return xrec, size


# speedup vs baseline: 1.7722x; 1.1577x over previous
import functools
import math

import jax
import jax.numpy as jnp
from jax.experimental import pallas as pl
from jax.experimental.pallas import tpu as pltpu

R_RATIO = 0.95
MARGIN = 0.5
_PREC = jax.lax.Precision.DEFAULT


def _iota_f32(shape, dim):
    return jax.lax.broadcasted_iota(jnp.int32, shape, dim).astype(jnp.float32)


def _iso_body(n_blk_ref, n_all_ref, x_all_ref, z_ref, *, block_i):
    ib = pl.program_id(1)
    nb = n_blk_ref[0, 0, :]
    na = n_all_ref[0, 0, :]
    xa = x_all_ref[0] / jnp.clip(na[:, None], 1e-12, None)
    T = xa.shape[0]
    xi = x_all_ref[0, pl.ds(ib * block_i, block_i), :] / jnp.clip(nb[:, None], 1e-12, None)
    sim = jax.lax.dot_general(xi, xa, (((1,), (1,)), ((), ())),
                              preferred_element_type=jnp.float32,
                              precision=_PREC)
    cnt = jnp.sum(jnp.where(sim > MARGIN, 1.0, -1.0), axis=1)
    ssum = jnp.sum(sim, axis=1)
    z_ref[0, 0, :] = cnt / T + ssum / T


def _iso_pass(x, n3, block_i=512):
    B, T, C = x.shape
    z = pl.pallas_call(
        functools.partial(_iso_body, block_i=block_i),
        grid=(B, T // block_i),
        in_specs=[
            pl.BlockSpec((1, 1, block_i), lambda b, i: (b, 0, i)),
            pl.BlockSpec((1, 1, T), lambda b, i: (b, 0, 0)),
            pl.BlockSpec((1, T, C), lambda b, i: (b, 0, 0)),
        ],
        out_specs=pl.BlockSpec((1, 1, block_i), lambda b, i: (b, 0, i)),
        out_shape=jax.ShapeDtypeStruct((B, 1, T), jnp.float32),
    )(n3, n3, x)
    return z


def _select_body(iso_col_ref, iso_row_ref, n_col_ref, x_ref, outpos_ref,
                 *, r, n_prot, kpad, chunk):
    T, C = x_ref[0].shape
    ir = iso_row_ref[0]
    ilane = _iota_f32((chunk, T), 1)

    rank_row = jnp.zeros((1, T), jnp.float32)
    rank_cols = []
    for cb in range(T // chunk):
        ic = iso_col_ref[0, pl.ds(cb * chunk, chunk), :]
        jsub = _iota_f32((chunk, T), 0) + float(cb * chunk)
        take = jnp.where(jnp.logical_or(ic < ir,
                                        jnp.logical_and(ic == ir, jsub < ilane)),
                         1.0, 0.0)
        rank_row = rank_row + jnp.sum(take, axis=0, keepdims=True)
        rank_cols.append((T - 1.0) - jnp.sum(take, axis=1, keepdims=True))

    c = jnp.zeros((1, T), jnp.float32)
    for cb in range(T // chunk):
        prot_col = jnp.where(rank_cols[cb] >= 2.0 * r, 1.0, 0.0)
        jsub = _iota_f32((chunk, T), 0) + float(cb * chunk)
        c = c + jnp.sum(jnp.where(jsub < ilane, prot_col, 0.0), axis=0,
                        keepdims=True)

    xn = x_ref[0] / jnp.clip(n_col_ref[0], 1e-12, None)
    kk = _iota_f32((kpad, T), 0)
    kvalid = kk < r
    rr = jnp.broadcast_to(rank_row, (kpad, T))
    ea = jnp.where(jnp.logical_and(rr == 2.0 * kk, kvalid), 1.0, 0.0)
    eb = jnp.where(jnp.logical_and(rr == 2.0 * kk + 1.0, kvalid), 1.0, 0.0)

    xa = jax.lax.dot_general(ea, xn, (((1,), (0,)), ((), ())),
                             preferred_element_type=jnp.float32, precision=_PREC)
    xb = jax.lax.dot_general(eb, xn, (((1,), (0,)), ((), ())),
                             preferred_element_type=jnp.float32, precision=_PREC)
    scores = jax.lax.dot_general(xa, xb, (((1,), (1,)), ((), ())),
                                 preferred_element_type=jnp.float32,
                                 precision=_PREC)
    jjk = _iota_f32((kpad, kpad), 1)
    scores_m = jnp.where(jjk < r, scores, -3.0)
    mx = jnp.max(scores_m, axis=1, keepdims=True)
    dst = jnp.min(jnp.where(scores_m == mx, jjk, float(kpad)), axis=1,
                  keepdims=True)

    dst_tok = jnp.sum(ea * dst, axis=0, keepdims=True)

    is_merged = rank_row < 2.0 * r
    is_b = jnp.logical_and(is_merged,
                           rank_row - 2.0 * jnp.floor(rank_row * 0.5) == 1.0)
    pos_b = n_prot + (rank_row - 1.0) * 0.5
    pos_a = n_prot + dst_tok
    outpos_ref[0] = jnp.where(is_merged, jnp.where(is_b, pos_b, pos_a), c)


def _select_pass(iso_col, iso_row, n_col, x, r, n_prot, kpad=128, chunk=512):
    B, T, C = x.shape
    return pl.pallas_call(
        functools.partial(_select_body, r=r, n_prot=n_prot, kpad=kpad,
                          chunk=chunk),
        grid=(B,),
        in_specs=[
            pl.BlockSpec((1, T, 1), lambda b: (b, 0, 0)),
            pl.BlockSpec((1, 1, T), lambda b: (b, 0, 0)),
            pl.BlockSpec((1, T, 1), lambda b: (b, 0, 0)),
            pl.BlockSpec((1, T, C), lambda b: (b, 0, 0)),
        ],
        out_specs=pl.BlockSpec((1, 1, T), lambda b: (b, 0, 0)),
        out_shape=jax.ShapeDtypeStruct((B, 1, T), jnp.float32),
    )(iso_col, iso_row, n_col, x)


def _merge_body(outpos_ref, w_ref, w_col_ref, x_ref, xrec_ref, size_ref,
                *, block_p):
    outpos = outpos_ref[0]
    w = w_ref[0]
    w_col = w_col_ref[0]
    x = x_ref[0]
    T = x.shape[0]
    pb = pl.program_id(1)
    pcol = _iota_f32((block_p, T), 0) + pb * block_p
    sel = jnp.broadcast_to(outpos, (block_p, T)) == pcol
    m01 = jnp.where(sel, 1.0, 0.0)
    y = x * w_col
    acc = jax.lax.dot_general(m01, y, (((1,), (0,)), ((), ())),
                              preferred_element_type=jnp.float32,
                              precision=_PREC)
    size = jnp.sum(jnp.where(sel, w, 0.0), axis=1, keepdims=True)
    xrec_ref[0] = acc / size
    size_ref[0] = size


def _merge_pass(outpos, iso_row, iso_col, x, n_out, block_p=256):
    B, T, C = x.shape
    n_blocks = (n_out + block_p - 1) // block_p
    xrec, size = pl.pallas_call(
        functools.partial(_merge_body, block_p=block_p),
        grid=(B, n_blocks),
        in_specs=[
            pl.BlockSpec((1, 1, T), lambda b, p: (b, 0, 0)),
            pl.BlockSpec((1, 1, T), lambda b, p: (b, 0, 0)),
            pl.BlockSpec((1, T, 1), lambda b, p: (b, 0, 0)),
            pl.BlockSpec((1, T, C), lambda b, p: (b, 0, 0)),
        ],
        out_specs=[
            pl.BlockSpec((1, block_p, C), lambda b, p: (b, p, 0)),
            pl.BlockSpec((1, block_p, 1), lambda b, p: (b, p, 0)),
        ],
        out_shape=[
            jax.ShapeDtypeStruct((B, n_out, C), jnp.float32),
            jax.ShapeDtypeStruct((B, n_out, 1), jnp.float32),
        ],
    )(outpos, iso_row, iso_col, x)
    return xrec, size


def kernel(x):
    B, T, C = x.shape
    r = math.floor(T - T * R_RATIO)
    n_prot = T - 2 * r
    n_out = n_prot + r

    nrm = jnp.linalg.norm(x, axis=-1, keepdims=True)
    n3 = nrm[..., 0].reshape(B, 1, T)
    z = _iso_pass(x, n3)

    iso = 1.0 - jax.nn.softmax(z[:, 0, :], axis=-1)
    iso_row = iso.reshape(B, 1, T)
    iso_col = iso.reshape(B, T, 1)

    outpos = _select_pass(iso_col, iso_row, nrm, x, r, n_prot)
    xrec, size = _merge_pass(outpos, iso_row, iso_col, x, n_out)
    return xrec, size


# speedup vs baseline: 1.9474x; 1.0989x over previous
import functools
import math

import jax
import jax.numpy as jnp
from jax.experimental import pallas as pl
from jax.experimental.pallas import tpu as pltpu

R_RATIO = 0.95
MARGIN = 0.5
_PREC = jax.lax.Precision.DEFAULT


def _iota_f32(shape, dim):
    return jax.lax.broadcasted_iota(jnp.int32, shape, dim).astype(jnp.float32)


def _iso_body(n_blk_ref, n_all_ref, x_all_ref, z_ref, *, block_i):
    ib = pl.program_id(1)
    nb = n_blk_ref[0, 0, :]
    na = n_all_ref[0, 0, :]
    xa = x_all_ref[0] / jnp.clip(na[:, None], 1e-12, None)
    T = xa.shape[0]
    xi = x_all_ref[0, pl.ds(ib * block_i, block_i), :] / jnp.clip(nb[:, None], 1e-12, None)
    sim = jax.lax.dot_general(xi, xa, (((1,), (1,)), ((), ())),
                              preferred_element_type=jnp.float32,
                              precision=_PREC)
    cnt = jnp.sum(jnp.where(sim > MARGIN, 1.0, -1.0), axis=1)
    ssum = jnp.sum(sim, axis=1)
    z_ref[0, 0, :] = cnt / T + ssum / T


def _iso_pass(x, n3, block_i=512):
    B, T, C = x.shape
    z = pl.pallas_call(
        functools.partial(_iso_body, block_i=block_i),
        grid=(B, T // block_i),
        in_specs=[
            pl.BlockSpec((1, 1, block_i), lambda b, i: (b, 0, i)),
            pl.BlockSpec((1, 1, T), lambda b, i: (b, 0, 0)),
            pl.BlockSpec((1, T, C), lambda b, i: (b, 0, 0)),
        ],
        out_specs=pl.BlockSpec((1, 1, block_i), lambda b, i: (b, 0, i)),
        out_shape=jax.ShapeDtypeStruct((B, 1, T), jnp.float32),
    )(n3, n3, x)
    return z


def _select_body(iso_col_ref, iso_row_ref, n_col_ref, x_ref, xrec_ref,
                 size_ref, *, r, n_prot, kpad, chunk, block_p):
    T, C = x_ref[0].shape
    ir = iso_row_ref[0]
    ilane = _iota_f32((chunk, T), 1)

    rank_row = jnp.zeros((1, T), jnp.float32)
    rank_cols = []
    for cb in range(T // chunk):
        ic = iso_col_ref[0, pl.ds(cb * chunk, chunk), :]
        jsub = _iota_f32((chunk, T), 0) + float(cb * chunk)
        take = jnp.where(jnp.logical_or(ic < ir,
                                        jnp.logical_and(ic == ir, jsub < ilane)),
                         1.0, 0.0)
        rank_row = rank_row + jnp.sum(take, axis=0, keepdims=True)
        rank_cols.append((T - 1.0) - jnp.sum(take, axis=1, keepdims=True))

    c = jnp.zeros((1, T), jnp.float32)
    for cb in range(T // chunk):
        prot_col = jnp.where(rank_cols[cb] >= 2.0 * r, 1.0, 0.0)
        jsub = _iota_f32((chunk, T), 0) + float(cb * chunk)
        c = c + jnp.sum(jnp.where(jsub < ilane, prot_col, 0.0), axis=0,
                        keepdims=True)

    xn = x_ref[0] / jnp.clip(n_col_ref[0], 1e-12, None)
    kk = _iota_f32((kpad, T), 0)
    kvalid = kk < r
    rr = jnp.broadcast_to(rank_row, (kpad, T))
    ea = jnp.where(jnp.logical_and(rr == 2.0 * kk, kvalid), 1.0, 0.0)
    eb = jnp.where(jnp.logical_and(rr == 2.0 * kk + 1.0, kvalid), 1.0, 0.0)

    xa = jax.lax.dot_general(ea, xn, (((1,), (0,)), ((), ())),
                             preferred_element_type=jnp.float32, precision=_PREC)
    xb = jax.lax.dot_general(eb, xn, (((1,), (0,)), ((), ())),
                             preferred_element_type=jnp.float32, precision=_PREC)
    scores = jax.lax.dot_general(xa, xb, (((1,), (1,)), ((), ())),
                                 preferred_element_type=jnp.float32,
                                 precision=_PREC)
    jjk = _iota_f32((kpad, kpad), 1)
    scores_m = jnp.where(jjk < r, scores, -3.0)
    mx = jnp.max(scores_m, axis=1, keepdims=True)
    dst = jnp.min(jnp.where(scores_m == mx, jjk, float(kpad)), axis=1,
                  keepdims=True)

    dst_tok = jnp.sum(ea * dst, axis=0, keepdims=True)

    is_merged = rank_row < 2.0 * r
    is_b = jnp.logical_and(is_merged,
                           rank_row - 2.0 * jnp.floor(rank_row * 0.5) == 1.0)
    pos_b = n_prot + (rank_row - 1.0) * 0.5
    pos_a = n_prot + dst_tok
    outpos = jnp.where(is_merged, jnp.where(is_b, pos_b, pos_a), c)

    iso_row = ir
    iso_col = iso_col_ref[0]
    y = x_ref[0] * iso_col
    n_out = n_prot + r
    for pb in range(0, n_out, block_p):
        bp = min(block_p, n_out - pb)
        pcol = _iota_f32((block_p, T), 0) + float(pb)
        sel = jnp.broadcast_to(outpos, (block_p, T)) == pcol
        m01 = jnp.where(sel, 1.0, 0.0)
        acc = jax.lax.dot_general(m01, y, (((1,), (0,)), ((), ())),
                                  preferred_element_type=jnp.float32,
                                  precision=_PREC)
        size = jnp.sum(jnp.where(sel, iso_row, 0.0), axis=1, keepdims=True)
        xrec_ref[0, pl.ds(pb, bp), :] = (acc / size)[:bp, :]
        size_ref[0, pl.ds(pb, bp), :] = size[:bp, :]


def _select_pass(iso_col, iso_row, n_col, x, r, n_prot, kpad=128, chunk=512,
                 block_p=256):
    B, T, C = x.shape
    n_out = n_prot + r
    return pl.pallas_call(
        functools.partial(_select_body, r=r, n_prot=n_prot, kpad=kpad,
                          chunk=chunk, block_p=block_p),
        grid=(B,),
        in_specs=[
            pl.BlockSpec((1, T, 1), lambda b: (b, 0, 0)),
            pl.BlockSpec((1, 1, T), lambda b: (b, 0, 0)),
            pl.BlockSpec((1, T, 1), lambda b: (b, 0, 0)),
            pl.BlockSpec((1, T, C), lambda b: (b, 0, 0)),
        ],
        out_specs=[
            pl.BlockSpec((1, n_out, C), lambda b: (b, 0, 0)),
            pl.BlockSpec((1, n_out, 1), lambda b: (b, 0, 0)),
        ],
        out_shape=[
            jax.ShapeDtypeStruct((B, n_out, C), jnp.float32),
            jax.ShapeDtypeStruct((B, n_out, 1), jnp.float32),
        ],
    )(iso_col, iso_row, n_col, x)


def kernel(x):
    B, T, C = x.shape
    r = math.floor(T - T * R_RATIO)
    n_prot = T - 2 * r
    n_out = n_prot + r

    nrm = jnp.linalg.norm(x, axis=-1, keepdims=True)
    n3 = nrm[..., 0].reshape(B, 1, T)
    z = _iso_pass(x, n3)

    iso = 1.0 - jax.nn.softmax(z[:, 0, :], axis=-1)
    iso_row = iso.reshape(B, 1, T)
    iso_col = iso.reshape(B, T, 1)

    xrec, size = _select_pass(iso_col, iso_row, nrm, x, r, n_prot)
    return xrec, size
